# baseline (speedup 1.0000x reference)
import functools

import jax
import jax.numpy as jnp
from jax import lax
from jax.experimental import pallas as pl
from jax.experimental.pallas import tpu as pltpu
from jax.experimental.pallas import tpu_sc as plsc

N = 50000
D = 128
E = 250000
NUM_REL = 3

N_PAD = 50176
STRIPE = N_PAD // 16
E_PAD = 262144
EROWS = E_PAD // 128
TROWS = EROWS // 16

CHUNK = 4096
NCHUNK = 13
DUMP = CHUNK
SEGROWS = 32
NSEG = TROWS // SEGROWS
CAP = SEGROWS * 128 + 16


def _sc_mesh():
    return plsc.VectorSubcoreMesh(core_axis_name="c", subcore_axis_name="s")


def _k1_body(idx_hbm, norms_hbm, hist, idx_v, ones_v, work_v, sem):
    cid = lax.axis_index("c")
    sid = lax.axis_index("s")

    z16 = jnp.zeros((16,), jnp.float32)
    o16 = jnp.ones((16,), jnp.float32)
    for j in range(8):
        ones_v[pl.ds(j * 16, 16)] = o16

    for k in range(3):
        hid = 2 * k + cid

        def fill_z(i, _):
            work_v[pl.ds(i * 16, 16)] = z16
            return 0

        lax.fori_loop(0, STRIPE // 16, fill_z, 0)
        pltpu.sync_copy(work_v, hist.at[pl.ds(sid * STRIPE, STRIPE)])
        plsc.subcore_barrier()

        row0 = pl.multiple_of(hid * EROWS + sid * TROWS, 8)
        pltpu.sync_copy(idx_hbm.at[pl.ds(row0, TROWS), :], idx_v)

        def scat(j, _):
            pltpu.sync_copy(ones_v, hist.at[idx_v.at[j]], add=True)
            return 0

        lax.fori_loop(0, TROWS, scat, 0)
        plsc.subcore_barrier()

        pltpu.sync_copy(hist.at[pl.ds(sid * STRIPE, STRIPE)], work_v)

        def norm_one(i, _):
            d = work_v[pl.ds(i * 16, 16)]
            ds_ = jnp.maximum(d, 1.0)
            bits = lax.bitcast_convert_type(ds_, jnp.int32)
            y = lax.bitcast_convert_type(
                jnp.full((16,), 0x5F3759DF, jnp.int32) - (bits >> 1), jnp.float32
            )
            half = 0.5 * ds_
            for _ in range(3):
                y = y * (1.5 - half * y * y)
            work_v[pl.ds(i * 16, 16)] = jnp.where(d > 0.5, y, 0.0)
            return 0

        lax.fori_loop(0, STRIPE // 16, norm_one, 0)
        off = pl.multiple_of(hid * N_PAD + sid * STRIPE, 8)
        pltpu.sync_copy(work_v, norms_hbm.at[pl.ds(off, STRIPE)])
        plsc.subcore_barrier()


def _k1(idx6):
    f = functools.partial(
        pl.kernel,
        out_type=jax.ShapeDtypeStruct((6 * N_PAD,), jnp.float32),
        mesh=_sc_mesh(),
        scratch_types=[
            pltpu.VMEM_SHARED((N_PAD,), jnp.float32),
            pltpu.VMEM((TROWS, 128), jnp.int32),
            pltpu.VMEM((128,), jnp.float32),
            pltpu.VMEM((STRIPE,), jnp.float32),
            pltpu.SemaphoreType.DMA,
        ],
    )(_k1_body)
    return f(idx6)


QCHUNK = N_PAD // 4
QSTRIPE = QCHUNK // 16
QDUMP = QCHUNK


def _mm_body(x_ref, w0_ref, w1_ref, w2_ref, ns0_ref, ns1_ref, ns2_ref,
             y0_ref, y1_ref, y2_ref):
    x = x_ref[...]
    hp = jax.lax.Precision.HIGHEST
    y0_ref[...] = jnp.dot(x * ns0_ref[...], w0_ref[...],
                          preferred_element_type=jnp.float32, precision=hp)
    y1_ref[...] = jnp.dot(x * ns1_ref[...], w1_ref[...],
                          preferred_element_type=jnp.float32, precision=hp)
    y2_ref[...] = jnp.dot(x * ns2_ref[...], w2_ref[...],
                          preferred_element_type=jnp.float32, precision=hp)


def _mm3(x, ws, nss):
    blk = pl.BlockSpec((1024, 128), lambda i: (i, 0))
    wspec = pl.BlockSpec((128, 128), lambda i: (0, 0))
    nspec = pl.BlockSpec((1024, 1), lambda i: (i, 0))
    return pl.pallas_call(
        _mm_body,
        grid=(N_PAD // 1024,),
        in_specs=[blk, wspec, wspec, wspec, nspec, nspec, nspec],
        out_specs=[blk, blk, blk],
        out_shape=[jax.ShapeDtypeStruct((N_PAD, D), jnp.float32)] * 3,
    )(x, *ws, *nss)


def _k3_body(y0, y1, y2, idx_hbm, agg0, agg1, agg2,
             acc, srcw, dstw, blkd, rb0, rb1, sem0, sem1):
    cid = lax.axis_index("c")
    sid = lax.axis_index("s")
    ys = [y0, y1, y2]
    aggs = [agg0, agg1, agg2]
    rbs = [rb0, rb1]
    sems = [sem0, sem1]

    z16 = jnp.zeros((16,), jnp.float32)
    dump16 = jnp.full((16,), QDUMP, jnp.int32)

    def fill_rb0_zero(i, _):
        for l in range(8):
            rb0[i, pl.ds(l * 16, 16)] = z16
        return 0

    for q2 in range(2):
        quarter = cid * 2 + q2
        base = quarter * QCHUNK
        for r in range(NUM_REL):
            yr = ys[r]
            sbase = 2 * r * EROWS + sid * TROWS
            dbase = (2 * r + 1) * EROWS + sid * TROWS

            lax.fori_loop(0, 64, fill_rb0_zero, 0)
            for m in range(12):
                pltpu.sync_copy(rb0, acc.at[pl.ds(sid * QSTRIPE + m * 64, 64), :])
            pltpu.sync_copy(rb0.at[pl.ds(0, 16), :],
                            acc.at[pl.ds(sid * QSTRIPE + 768, 16), :])
            plsc.subcore_barrier()

            def seg_body(seg, _):
                srow = pl.multiple_of(sbase + seg * SEGROWS, 8)
                drow = pl.multiple_of(dbase + seg * SEGROWS, 8)
                pltpu.sync_copy(idx_hbm.at[pl.ds(srow, SEGROWS), :], srcw)
                pltpu.sync_copy(idx_hbm.at[pl.ds(drow, SEGROWS), :], dstw)

                def dl_row(j, _):
                    for l in range(8):
                        d = dstw[j, pl.ds(l * 16, 16)]
                        inq = (d >= base) & (d < base + QCHUNK)
                        dstw[j, pl.ds(l * 16, 16)] = jnp.where(
                            inq, d - base, dump16
                        )
                    return 0

                lax.fori_loop(0, SEGROWS, dl_row, 0)

                def issue(k, par):
                    j2 = k >> 1
                    h = k & 1
                    return pltpu.async_copy(
                        yr.at[srcw.at[j2, pl.ds(h * 64, 64)]],
                        rbs[par], sems[par],
                    )

                issue(0, 0)
                issue(1, 1)

                def pair(p, _):
                    for par in range(2):
                        k = 2 * p + par
                        pltpu.make_async_copy(
                            yr.at[srcw.at[0, pl.ds(0, 64)]],
                            rbs[par], sems[par],
                        ).wait()
                        j2 = k >> 1
                        for l in range(4):
                            blkd[pl.ds(l * 16, 16)] = dstw[
                                j2, pl.ds(par * 64 + l * 16, 16)
                            ]
                        pltpu.sync_copy(rbs[par], acc.at[blkd], add=True)
                        kn = jnp.minimum(k + 2, 63)
                        issue(kn, par)
                    return 0

                lax.fori_loop(0, SEGROWS, pair, 0)
                for par in range(2):
                    pltpu.make_async_copy(
                        yr.at[srcw.at[0, pl.ds(0, 64)]],
                        rbs[par], sems[par],
                    ).wait()
                return 0

            lax.fori_loop(0, NSEG, seg_body, 0)
            plsc.subcore_barrier()

            for m in range(13):
                nrows = 64 if m < 12 else 16
                a0 = sid * QSTRIPE + m * 64
                g0 = pl.multiple_of(base + a0, 8)
                pltpu.sync_copy(acc.at[pl.ds(a0, nrows), :],
                                rb0.at[pl.ds(0, nrows), :])
                pltpu.sync_copy(rb0.at[pl.ds(0, nrows), :],
                                aggs[r].at[pl.ds(g0, nrows), :])
            plsc.subcore_barrier()


def _k3(ys, idx6):
    f = functools.partial(
        pl.kernel,
        out_type=[jax.ShapeDtypeStruct((N_PAD, 128), jnp.float32)] * 3,
        mesh=_sc_mesh(),
        scratch_types=[
            pltpu.VMEM_SHARED((QCHUNK + 8, 128), jnp.float32),
            pltpu.VMEM((SEGROWS, 128), jnp.int32),
            pltpu.VMEM((SEGROWS, 128), jnp.int32),
            pltpu.VMEM((64,), jnp.int32),
            pltpu.VMEM((64, 128), jnp.float32),
            pltpu.VMEM((64, 128), jnp.float32),
            pltpu.SemaphoreType.DMA,
            pltpu.SemaphoreType.DMA,
        ],
    )(_k3_body)
    return f(*ys, idx6)


def _comb_body(a0_ref, a1_ref, a2_ref, nd0_ref, nd1_ref, nd2_ref, b_ref, o_ref):
    o_ref[...] = (
        a0_ref[...] * nd0_ref[...]
        + a1_ref[...] * nd1_ref[...]
        + a2_ref[...] * nd2_ref[...]
        + b_ref[...]
    )


def _comb(aggs, nds, bias):
    blk = pl.BlockSpec((1024, 128), lambda i: (i, 0))
    nspec = pl.BlockSpec((1024, 1), lambda i: (i, 0))
    bspec = pl.BlockSpec((1, 128), lambda i: (0, 0))
    return pl.pallas_call(
        _comb_body,
        grid=(N_PAD // 1024,),
        in_specs=[blk, blk, blk, nspec, nspec, nspec, bspec],
        out_specs=blk,
        out_shape=jax.ShapeDtypeStruct((N_PAD, D), jnp.float32),
    )(*aggs, *nds, bias)


def kernel(h, edge_index_rel0, edge_index_rel1, edge_index_rel2, W0, b0, W1, b1, W2, b2):
    eis = [edge_index_rel0, edge_index_rel1, edge_index_rel2]
    pad = jnp.full((E_PAD - E,), N, jnp.int32)
    srcs = [jnp.concatenate([ei[0].astype(jnp.int32), pad]) for ei in eis]
    dsts = [jnp.concatenate([ei[1].astype(jnp.int32), pad]) for ei in eis]
    idx6 = jnp.concatenate(
        [srcs[0], dsts[0], srcs[1], dsts[1], srcs[2], dsts[2]]
    ).reshape(6 * EROWS, 128)

    norms = _k1(idx6).reshape(6, N_PAD)
    nss = [norms[2 * r].reshape(N_PAD, 1) for r in range(NUM_REL)]
    nds = [norms[2 * r + 1].reshape(N_PAD, 1) for r in range(NUM_REL)]

    h_pad = jnp.concatenate([h, jnp.zeros((N_PAD - N, D), jnp.float32)])
    ys = _mm3(h_pad, [W0, W1, W2], nss)

    aggs = _k3(ys, idx6)
    bias = (b0 + b1 + b2).astype(jnp.float32).reshape(1, D)
    out = _comb(aggs, nds, bias)
    return out[:N]
